# baseline (speedup 1.0000x reference)
import functools

import jax
import jax.numpy as jnp
from jax import lax
from jax.experimental import pallas as pl
from jax.experimental.pallas import tpu as pltpu
from jax.experimental.pallas import tpu_sc as plsc

_LANES = 16
_NC = 2
_NS = 16
_NW = _NC * _NS
_CHUNK = 128
_EPS = 1e-12


def _allsum16(x):
    idx = jnp.arange(_LANES, dtype=jnp.int32)
    for shift in (8, 4, 2, 1):
        perm = (idx + shift) & (_LANES - 1)
        x = x + x.at[perm].get(mode="promise_in_bounds")
    return x


def _rsqrt16(x):
    i = lax.bitcast_convert_type(x, jnp.int32)
    i = jnp.int32(0x5F3759DF) - (i >> 1)
    y = lax.bitcast_convert_type(i, jnp.float32)
    for _ in range(2):
        y = y * (1.5 - 0.5 * x * y * y)
    return y


@functools.lru_cache(maxsize=None)
def _make_sc_call(n_rows: int, hid: int):
    assert hid % _LANES == 0
    nblk = hid // _LANES
    assert n_rows % (_NW * _CHUNK) == 0
    rpw = n_rows // _NW
    nch = rpw // _CHUNK
    assert nch % 2 == 0

    mesh = plsc.VectorSubcoreMesh(core_axis_name="c", subcore_axis_name="s")

    @functools.partial(
        pl.kernel,
        mesh=mesh,
        out_type=jax.ShapeDtypeStruct((n_rows, hid), jnp.float32),
        scratch_types=[
            pltpu.VMEM((rpw,), jnp.int32),
            pltpu.VMEM((rpw,), jnp.int32),
            pltpu.VMEM((2, _CHUNK, hid), jnp.float32),
            pltpu.VMEM((2, _CHUNK, hid), jnp.float32),
            pltpu.VMEM((2, _CHUNK, hid), jnp.float32),
            pltpu.VMEM((2, hid), jnp.float32),
            pltpu.SemaphoreType.DMA,
            pltpu.SemaphoreType.DMA,
            pltpu.SemaphoreType.DMA,
            pltpu.SemaphoreType.DMA,
            pltpu.SemaphoreType.DMA,
            pltpu.SemaphoreType.DMA,
        ],
    )
    def sc_call(ids_hbm, cidx_hbm, word_hbm, combo_hbm, gamma_hbm, beta_hbm,
                out_hbm, widx_v, cidx_v, wbuf, cbuf, obuf, gb_v,
                sw0, sw1, sk0, sk1, so0, so1):
        sw = [sw0, sw1]
        sk = [sk0, sk1]
        so = [so0, so1]
        wid = lax.axis_index("s") * _NC + lax.axis_index("c")
        base0 = wid * rpw
        p0 = pltpu.make_async_copy(ids_hbm.at[pl.ds(base0, rpw)], widx_v, sw0)
        p1 = pltpu.make_async_copy(cidx_hbm.at[pl.ds(base0, rpw)], cidx_v, sk0)
        p2 = pltpu.make_async_copy(gamma_hbm, gb_v.at[0], so0)
        p3 = pltpu.make_async_copy(beta_hbm, gb_v.at[1], so1)
        for c in (p0, p1, p2, p3):
            c.start()
        for c in (p0, p1, p2, p3):
            c.wait()
        gvec = [gb_v[0, pl.ds(j * _LANES, _LANES)] for j in range(nblk)]
        bvec = [gb_v[1, pl.ds(j * _LANES, _LANES)] for j in range(nblk)]

        def g_copies(s, g):
            return (
                pltpu.make_async_copy(
                    word_hbm.at[widx_v.at[pl.ds(g * _CHUNK, _CHUNK)]],
                    wbuf.at[s], sw[s]),
                pltpu.make_async_copy(
                    combo_hbm.at[cidx_v.at[pl.ds(g * _CHUNK, _CHUNK)]],
                    cbuf.at[s], sk[s]),
            )

        def o_copy(s, g):
            return pltpu.make_async_copy(
                obuf.at[s], out_hbm.at[pl.ds(base0 + g * _CHUNK, _CHUNK)],
                so[s])

        def g_start(s, g):
            for c in g_copies(s, g):
                c.start()

        def g_wait(s, g):
            for c in g_copies(s, g):
                c.wait()

        def compute(s):
            wb = wbuf.at[s]
            cb = cbuf.at[s]
            ob = obuf.at[s]

            @plsc.parallel_loop(0, _CHUNK, step=1, unroll=4)
            def row_body(r):
                vs = []
                acc_s = jnp.zeros((_LANES,), jnp.float32)
                acc_q = jnp.zeros((_LANES,), jnp.float32)
                for j in range(nblk):
                    v = (wb[r, pl.ds(j * _LANES, _LANES)]
                         + cb[r, pl.ds(j * _LANES, _LANES)])
                    vs.append(v)
                    acc_s = acc_s + v
                    acc_q = acc_q + v * v
                inv_n = jnp.float32(1.0 / hid)
                meanv = _allsum16(acc_s) * inv_n
                varv = _allsum16(acc_q) * inv_n - meanv * meanv
                rstd = _rsqrt16(varv + _EPS)
                for j in range(nblk):
                    o = (vs[j] - meanv) * rstd * gvec[j] + bvec[j]
                    ob[r, pl.ds(j * _LANES, _LANES)] = o

        g_start(0, 0)

        def body(i, carry):
            ca = 2 * i
            cb_ = 2 * i + 1
            g_start(1, cb_)
            g_wait(0, ca)

            @pl.when(i > 0)
            def _():
                o_copy(0, ca - 2).wait()

            compute(0)
            o_copy(0, ca).start()

            @pl.when(i + 1 < nch // 2)
            def _():
                g_start(0, ca + 2)

            g_wait(1, cb_)

            @pl.when(i > 0)
            def _():
                o_copy(1, cb_ - 2).wait()

            compute(1)
            o_copy(1, cb_).start()
            return carry

        lax.fori_loop(0, nch // 2, body, 0)
        o_copy(0, nch - 2).wait()
        o_copy(1, nch - 1).wait()

    return sc_call


def kernel(input_ids, token_type_ids, word_emb, type_emb, pos_emb, gamma, beta):
    B, L = input_ids.shape
    vocab, hid = word_emb.shape
    n = B * L
    ids_flat = input_ids.reshape(n).astype(jnp.int32)
    pos_ids = jnp.arange(L, dtype=jnp.int32)
    cidx = (token_type_ids.astype(jnp.int32) * L
            + pos_ids[None, :]).reshape(n)
    combo = (type_emb[:, None, :] + pos_emb[None, :L, :]).reshape(-1, hid)
    sc_call = _make_sc_call(n, hid)
    out = sc_call(ids_flat, cidx, word_emb, combo,
                  gamma.astype(jnp.float32), beta.astype(jnp.float32))
    return out.reshape(B, L, hid)


# speedup vs baseline: 1.0150x; 1.0150x over previous
import functools

import jax
import jax.numpy as jnp
from jax import lax
from jax.experimental import pallas as pl
from jax.experimental.pallas import tpu as pltpu
from jax.experimental.pallas import tpu_sc as plsc

_LANES = 16
_NC = 2
_NS = 16
_NW = _NC * _NS
_CHUNK = 128
_EPS = 1e-12


def _allsum16(x):
    idx = jnp.arange(_LANES, dtype=jnp.int32)
    for shift in (8, 4, 2, 1):
        perm = (idx + shift) & (_LANES - 1)
        x = x + x.at[perm].get(mode="promise_in_bounds")
    return x


def _rsqrt16(x):
    i = lax.bitcast_convert_type(x, jnp.int32)
    i = jnp.int32(0x5F3759DF) - (i >> 1)
    y = lax.bitcast_convert_type(i, jnp.float32)
    for _ in range(2):
        y = y * (1.5 - 0.5 * x * y * y)
    return y


@functools.lru_cache(maxsize=None)
def _make_sc_call(n_rows: int, hid: int):
    assert hid % _LANES == 0
    nblk = hid // _LANES
    assert n_rows % (_NW * _CHUNK) == 0
    rpw = n_rows // _NW
    nch = rpw // _CHUNK
    assert nch % 2 == 0

    mesh = plsc.VectorSubcoreMesh(core_axis_name="c", subcore_axis_name="s")

    @functools.partial(
        pl.kernel,
        mesh=mesh,
        out_type=jax.ShapeDtypeStruct((n_rows, hid), jnp.float32),
        scratch_types=[
            pltpu.VMEM((rpw,), jnp.int32),
            pltpu.VMEM((rpw,), jnp.int32),
            pltpu.VMEM((2, _CHUNK, hid), jnp.float32),
            pltpu.VMEM((2, _CHUNK, hid), jnp.float32),
            pltpu.VMEM((2, _CHUNK, hid), jnp.float32),
            pltpu.VMEM((2, hid), jnp.float32),
            pltpu.SemaphoreType.DMA,
            pltpu.SemaphoreType.DMA,
            pltpu.SemaphoreType.DMA,
            pltpu.SemaphoreType.DMA,
            pltpu.SemaphoreType.DMA,
            pltpu.SemaphoreType.DMA,
            pltpu.SemaphoreType.DMA,
            pltpu.SemaphoreType.DMA,
        ],
    )
    def sc_call(ids_hbm, cidx_hbm, word_hbm, combo_hbm, gamma_hbm, beta_hbm,
                out_hbm, widx_v, cidx_v, wbuf, cbuf, obuf, gb_v,
                sw0, sw1, sx0, sx1, sk0, sk1, so0, so1):
        sw = [sw0, sw1]
        sx = [sx0, sx1]
        sk = [sk0, sk1]
        so = [so0, so1]
        wid = lax.axis_index("s") * _NC + lax.axis_index("c")
        base0 = wid * rpw
        pltpu.sync_copy(ids_hbm.at[pl.ds(base0, rpw)], widx_v)
        pltpu.sync_copy(cidx_hbm.at[pl.ds(base0, rpw)], cidx_v)
        pltpu.sync_copy(gamma_hbm, gb_v.at[0])
        pltpu.sync_copy(beta_hbm, gb_v.at[1])
        gvec = [gb_v[0, pl.ds(j * _LANES, _LANES)] for j in range(nblk)]
        bvec = [gb_v[1, pl.ds(j * _LANES, _LANES)] for j in range(nblk)]

        half = _CHUNK // 2

        def g_copies(s, g):
            return (
                pltpu.make_async_copy(
                    word_hbm.at[widx_v.at[pl.ds(g * _CHUNK, half)]],
                    wbuf.at[s].at[pl.ds(0, half)], sw[s]),
                pltpu.make_async_copy(
                    word_hbm.at[widx_v.at[pl.ds(g * _CHUNK + half, half)]],
                    wbuf.at[s].at[pl.ds(half, half)], sx[s]),
                pltpu.make_async_copy(
                    combo_hbm.at[cidx_v.at[pl.ds(g * _CHUNK, _CHUNK)]],
                    cbuf.at[s], sk[s]),
            )

        def o_copy(s, g):
            return pltpu.make_async_copy(
                obuf.at[s], out_hbm.at[pl.ds(base0 + g * _CHUNK, _CHUNK)],
                so[s])

        def g_start(s, g):
            for c in g_copies(s, g):
                c.start()

        def g_wait(s, g):
            for c in g_copies(s, g):
                c.wait()

        def compute(s):
            wb = wbuf.at[s]
            cb = cbuf.at[s]
            ob = obuf.at[s]

            def row_body(r2, rc):
                for u in range(2):
                    r = r2 * 2 + u
                    vs = []
                    acc_s = jnp.zeros((_LANES,), jnp.float32)
                    acc_q = jnp.zeros((_LANES,), jnp.float32)
                    for j in range(nblk):
                        v = (wb[r, pl.ds(j * _LANES, _LANES)]
                             + cb[r, pl.ds(j * _LANES, _LANES)])
                        vs.append(v)
                        acc_s = acc_s + v
                        acc_q = acc_q + v * v
                    inv_n = jnp.float32(1.0 / hid)
                    meanv = _allsum16(acc_s) * inv_n
                    varv = _allsum16(acc_q) * inv_n - meanv * meanv
                    rstd = _rsqrt16(varv + _EPS)
                    for j in range(nblk):
                        o = (vs[j] - meanv) * rstd * gvec[j] + bvec[j]
                        ob[r, pl.ds(j * _LANES, _LANES)] = o
                return rc

            lax.fori_loop(0, _CHUNK // 2, row_body, 0)

        g_start(0, 0)

        def body(i, carry):
            ca = 2 * i
            cb_ = 2 * i + 1
            g_start(1, cb_)
            g_wait(0, ca)

            @pl.when(i > 0)
            def _():
                o_copy(0, ca - 2).wait()

            compute(0)
            o_copy(0, ca).start()

            @pl.when(i + 1 < nch // 2)
            def _():
                g_start(0, ca + 2)

            g_wait(1, cb_)

            @pl.when(i > 0)
            def _():
                o_copy(1, cb_ - 2).wait()

            compute(1)
            o_copy(1, cb_).start()
            return carry

        lax.fori_loop(0, nch // 2, body, 0)
        o_copy(0, nch - 2).wait()
        o_copy(1, nch - 1).wait()

    return sc_call


def kernel(input_ids, token_type_ids, word_emb, type_emb, pos_emb, gamma, beta):
    B, L = input_ids.shape
    vocab, hid = word_emb.shape
    n = B * L
    ids_flat = input_ids.reshape(n).astype(jnp.int32)
    pos_ids = jnp.arange(L, dtype=jnp.int32)
    cidx = (token_type_ids.astype(jnp.int32) * L
            + pos_ids[None, :]).reshape(n)
    combo = (type_emb[:, None, :] + pos_emb[None, :L, :]).reshape(-1, hid)
    sc_call = _make_sc_call(n, hid)
    out = sc_call(ids_flat, cidx, word_emb, combo,
                  gamma.astype(jnp.float32), beta.astype(jnp.float32))
    return out.reshape(B, L, hid)


# speedup vs baseline: 1.0166x; 1.0016x over previous
import functools

import jax
import jax.numpy as jnp
from jax import lax
from jax.experimental import pallas as pl
from jax.experimental.pallas import tpu as pltpu
from jax.experimental.pallas import tpu_sc as plsc

_LANES = 16
_NC = 2
_NS = 16
_NW = _NC * _NS
_CHUNK = 128
_EPS = 1e-12


def _allsum16(x):
    idx = jnp.arange(_LANES, dtype=jnp.int32)
    for shift in (8, 4, 2, 1):
        perm = (idx + shift) & (_LANES - 1)
        x = x + x.at[perm].get(mode="promise_in_bounds")
    return x


def _rsqrt16(x):
    i = lax.bitcast_convert_type(x, jnp.int32)
    i = jnp.int32(0x5F3759DF) - (i >> 1)
    y = lax.bitcast_convert_type(i, jnp.float32)
    for _ in range(2):
        y = y * (1.5 - 0.5 * x * y * y)
    return y


@functools.lru_cache(maxsize=None)
def _make_sc_call(n_rows: int, hid: int):
    assert hid % _LANES == 0
    nblk = hid // _LANES
    assert n_rows % (_NW * _CHUNK) == 0
    rpw = n_rows // _NW
    nch = rpw // _CHUNK
    assert nch % 2 == 0

    mesh = plsc.VectorSubcoreMesh(core_axis_name="c", subcore_axis_name="s")

    @functools.partial(
        pl.kernel,
        mesh=mesh,
        out_type=jax.ShapeDtypeStruct((n_rows, hid), jnp.float32),
        scratch_types=[
            pltpu.VMEM((rpw,), jnp.int32),
            pltpu.VMEM((rpw,), jnp.int32),
            pltpu.VMEM((2, _CHUNK, hid), jnp.float32),
            pltpu.VMEM((2, _CHUNK, hid), jnp.float32),
            pltpu.VMEM((2, _CHUNK, hid), jnp.float32),
            pltpu.VMEM((2, hid), jnp.float32),
            pltpu.SemaphoreType.DMA,
            pltpu.SemaphoreType.DMA,
            pltpu.SemaphoreType.DMA,
            pltpu.SemaphoreType.DMA,
            pltpu.SemaphoreType.DMA,
            pltpu.SemaphoreType.DMA,
        ],
    )
    def sc_call(ids_hbm, cidx_hbm, word_hbm, combo_hbm, gamma_hbm, beta_hbm,
                out_hbm, widx_v, cidx_v, wbuf, cbuf, obuf, gb_v,
                sw0, sw1, sk0, sk1, so0, so1):
        sw = [sw0, sw1]
        sk = [sk0, sk1]
        so = [so0, so1]
        wid = lax.axis_index("s") * _NC + lax.axis_index("c")
        base0 = wid * rpw
        pltpu.sync_copy(ids_hbm.at[pl.ds(base0, rpw)], widx_v)
        pltpu.sync_copy(cidx_hbm.at[pl.ds(base0, rpw)], cidx_v)
        pltpu.sync_copy(gamma_hbm, gb_v.at[0])
        pltpu.sync_copy(beta_hbm, gb_v.at[1])
        gvec = [gb_v[0, pl.ds(j * _LANES, _LANES)] for j in range(nblk)]
        bvec = [gb_v[1, pl.ds(j * _LANES, _LANES)] for j in range(nblk)]

        def g_copies(s, g):
            return (
                pltpu.make_async_copy(
                    word_hbm.at[widx_v.at[pl.ds(g * _CHUNK, _CHUNK)]],
                    wbuf.at[s], sw[s]),
                pltpu.make_async_copy(
                    combo_hbm.at[cidx_v.at[pl.ds(g * _CHUNK, _CHUNK)]],
                    cbuf.at[s], sk[s]),
            )

        def o_copy(s, g):
            return pltpu.make_async_copy(
                obuf.at[s], out_hbm.at[pl.ds(base0 + g * _CHUNK, _CHUNK)],
                so[s])

        def g_start(s, g):
            for c in g_copies(s, g):
                c.start()

        def g_wait(s, g):
            for c in g_copies(s, g):
                c.wait()

        def compute(s):
            wb = wbuf.at[s]
            cb = cbuf.at[s]
            ob = obuf.at[s]

            def row_body(r2, rc):
                for u in range(2):
                    r = r2 * 2 + u
                    vs = []
                    acc_s = jnp.zeros((_LANES,), jnp.float32)
                    acc_q = jnp.zeros((_LANES,), jnp.float32)
                    for j in range(nblk):
                        v = (wb[r, pl.ds(j * _LANES, _LANES)]
                             + cb[r, pl.ds(j * _LANES, _LANES)])
                        vs.append(v)
                        acc_s = acc_s + v
                        acc_q = acc_q + v * v
                    inv_n = jnp.float32(1.0 / hid)
                    meanv = _allsum16(acc_s) * inv_n
                    varv = _allsum16(acc_q) * inv_n - meanv * meanv
                    rstd = _rsqrt16(varv + _EPS)
                    for j in range(nblk):
                        o = (vs[j] - meanv) * rstd * gvec[j] + bvec[j]
                        ob[r, pl.ds(j * _LANES, _LANES)] = o
                return rc

            lax.fori_loop(0, _CHUNK // 2, row_body, 0)

        g_start(0, 0)

        def body(i, carry):
            ca = 2 * i
            cb_ = 2 * i + 1
            g_start(1, cb_)
            g_wait(0, ca)

            @pl.when(i > 0)
            def _():
                o_copy(0, ca - 2).wait()

            compute(0)
            o_copy(0, ca).start()

            @pl.when(i + 1 < nch // 2)
            def _():
                g_start(0, ca + 2)

            g_wait(1, cb_)

            @pl.when(i > 0)
            def _():
                o_copy(1, cb_ - 2).wait()

            compute(1)
            o_copy(1, cb_).start()
            return carry

        lax.fori_loop(0, nch // 2, body, 0)
        o_copy(0, nch - 2).wait()
        o_copy(1, nch - 1).wait()

    return sc_call


def kernel(input_ids, token_type_ids, word_emb, type_emb, pos_emb, gamma, beta):
    B, L = input_ids.shape
    vocab, hid = word_emb.shape
    n = B * L
    ids_flat = input_ids.reshape(n).astype(jnp.int32)
    pos_ids = jnp.arange(L, dtype=jnp.int32)
    cidx = (token_type_ids.astype(jnp.int32) * L
            + pos_ids[None, :]).reshape(n)
    combo = (type_emb[:, None, :] + pos_emb[None, :L, :]).reshape(-1, hid)
    sc_call = _make_sc_call(n, hid)
    out = sc_call(ids_flat, cidx, word_emb, combo,
                  gamma.astype(jnp.float32), beta.astype(jnp.float32))
    return out.reshape(B, L, hid)
